# Initial kernel scaffold; baseline (speedup 1.0000x reference)
import jax
import jax.numpy as jnp
from jax.experimental import pallas as pl


def kernel(updates, mask):
    raise NotImplementedError("write your pallas kernel here")


# speedup vs baseline: 42.6395x; 42.6395x over previous
import functools

import jax
import jax.numpy as jnp
from jax import lax
from jax.experimental import pallas as pl
from jax.experimental.pallas import tpu as pltpu
from jax.experimental.pallas import tpu_sc as plsc

_NC, _NS, _L = 2, 16, 16
_NW = _NC * _NS


def _unpool_planes(vals_t, mask_t, n, p):
    rows = vals_t.shape[0]
    assert rows % (2 * _NW) == 0
    items2 = rows // _NW // 2
    chunks = 8
    ch = p // chunks
    zun = 8
    sun = 8
    assert ch % (_L * zun) == 0 and n % (_L * sun) == 0

    mesh = plsc.VectorSubcoreMesh(
        core_axis_name="c", subcore_axis_name="s",
        num_cores=_NC, num_subcores=_NS,
    )

    @functools.partial(
        pl.kernel,
        out_type=jax.ShapeDtypeStruct((rows, p), jnp.float32),
        mesh=mesh,
        compiler_params=pltpu.CompilerParams(needs_layout_passes=False),
        scratch_types=[
            pltpu.VMEM((n,), jnp.float32),
            pltpu.VMEM((n,), jnp.int32),
            pltpu.VMEM((n,), jnp.float32),
            pltpu.VMEM((n,), jnp.int32),
            pltpu.VMEM((p,), jnp.float32),
            pltpu.SemaphoreType.DMA,
            pltpu.SemaphoreType.DMA,
            pltpu.SemaphoreType.DMA,
            pltpu.SemaphoreType.DMA,
        ],
    )
    def k(vals_hbm, mask_hbm, out_hbm, va, ma, vb, mb, acc, sia, sib, so0, so1):
        wid = lax.axis_index("s") * _NC + lax.axis_index("c")

        def start_in(row, vbuf, mbuf, sem):
            pltpu.make_async_copy(vals_hbm.at[row], vbuf, sem).start()
            pltpu.make_async_copy(mask_hbm.at[row], mbuf, sem).start()

        def wait_in(vbuf, mbuf, sem):
            pltpu.make_async_copy(vals_hbm.at[0], vbuf, sem).wait()
            pltpu.make_async_copy(mask_hbm.at[0], mbuf, sem).wait()

        zv = jnp.zeros((_L,), jnp.float32)

        def zero_chunk(base):
            def zb(i, c):
                off = base + i * (_L * zun)
                for u in range(zun):
                    acc[pl.ds(off + u * _L, _L)] = zv
                return c
            lax.fori_loop(0, ch // (_L * zun), zb, 0)

        def scatter_item(vbuf, mbuf):
            def sb(i, c):
                base = i * (_L * sun)
                for u in range(sun):
                    m = mbuf[pl.ds(base + u * _L, _L)]
                    x = (m >> 6).astype(jnp.float32)
                    idx = ((x + 0.5) * (1.0 / 3.0)).astype(jnp.int32)
                    v = vbuf[pl.ds(base + u * _L, _L)]
                    plsc.addupdate_scatter(acc, [idx], v)
                return c
            lax.fori_loop(0, n // (_L * sun), sb, 0)

        def drain_item(row):
            sems = (so0, so1)
            cps = []
            for j in range(chunks):
                cp = pltpu.make_async_copy(
                    acc.at[pl.ds(j * ch, ch)],
                    out_hbm.at[row, pl.ds(j * ch, ch)],
                    sems[j % 2],
                )
                cp.start()
                if j >= 1:
                    cps[j - 1].wait()
                    zero_chunk((j - 1) * ch)
                cps.append(cp)
            cps[-1].wait()
            zero_chunk((chunks - 1) * ch)

        start_in(wid, va, ma, sia)
        for j in range(chunks):
            zero_chunk(j * ch)

        def body(i2, c):
            row_a = (2 * i2) * _NW + wid
            row_b = row_a + _NW
            start_in(row_b, vb, mb, sib)
            wait_in(va, ma, sia)
            scatter_item(va, ma)
            drain_item(row_a)

            @pl.when(i2 + 1 < items2)
            def _():
                start_in(row_b + _NW, va, ma, sia)

            wait_in(vb, mb, sib)
            scatter_item(vb, mb)
            drain_item(row_b)
            return c

        lax.fori_loop(0, items2, body, 0)

    return k(vals_t, mask_t)


def kernel(updates, mask):
    b, h, w, c = updates.shape
    n = h * w
    hout, wout = 2 * h, 2 * w
    p = hout * wout
    assert c == 192
    vals_t = updates.reshape(b, n, c).transpose(0, 2, 1).reshape(b * c, n)
    mask_t = mask.astype(jnp.int32).reshape(b, n, c).transpose(0, 2, 1)
    mask_t = mask_t.reshape(b * c, n)
    out_t = _unpool_planes(vals_t, mask_t, n, p)
    return out_t.reshape(b, c, hout, wout).transpose(0, 2, 3, 1)
